# baseline (speedup 1.0000x reference)
import functools

import jax
import jax.numpy as jnp
from jax import lax
from jax.experimental import pallas as pl
from jax.experimental.pallas import tpu as pltpu
from jax.experimental.pallas import tpu_sc as plsc

N = 2048
M = 64
CH = 256
NCH = N // CH
CHW = CH * M


def _shift_body(v_ref, w_ref):
    v = v_ref[...]
    w_ref[...] = v - jnp.min(v, axis=1, keepdims=True) + 1.0


def _tc_shift(v):
    return pl.pallas_call(
        _shift_body,
        out_shape=jax.ShapeDtypeStruct((N, M), jnp.float32),
    )(v)


def _sc_scan_body(w_hbm, out_hbm, inb0, inb1, outb0, outb1, sin, sout):
    wid = lax.axis_index("s") * 2 + lax.axis_index("c")

    @pl.when(wid == 0)
    def _():
        ones = jnp.ones((16,), jnp.float32)
        c = (ones, ones, ones, ones)
        lane = lax.iota(jnp.int32, 16)
        p8 = lane ^ 8
        p4 = lane ^ 4
        p2 = lane ^ 2
        p1 = lane ^ 1

        inbufs = (inb0, inb1)
        outbufs = (outb0, outb1)

        def make_row_step(inb, outb):
            def row_step(i, c4):
                c0, c1, c2, c3 = c4
                e0 = jnp.exp(inb[i, pl.ds(0, 16)] * c0)
                e1 = jnp.exp(inb[i, pl.ds(16, 16)] * c1)
                e2 = jnp.exp(inb[i, pl.ds(32, 16)] * c2)
                e3 = jnp.exp(inb[i, pl.ds(48, 16)] * c3)
                s = (e0 + e1) + (e2 + e3)
                s = s + s.at[p8].get(mode="promise_in_bounds")
                s = s + s.at[p4].get(mode="promise_in_bounds")
                s = s + s.at[p2].get(mode="promise_in_bounds")
                s = s + s.at[p1].get(mode="promise_in_bounds")
                inv = 1.0 / s
                y0 = e0 * inv
                y1 = e1 * inv
                y2 = e2 * inv
                y3 = e3 * inv
                outb[i, pl.ds(0, 16)] = y0
                outb[i, pl.ds(16, 16)] = y1
                outb[i, pl.ds(32, 16)] = y2
                outb[i, pl.ds(48, 16)] = y3
                return (c0 - c0 * y0, c1 - c1 * y1, c2 - c2 * y2, c3 - c3 * y3)
            return row_step

        cp0 = pltpu.make_async_copy(w_hbm.at[pl.ds(0, CH)], inb0, sin)
        cp0.start()
        cp0.wait()

        for g in range(NCH):
            inb = inbufs[g % 2]
            outb = outbufs[g % 2]
            if g + 1 < NCH:
                nxt = pltpu.make_async_copy(
                    w_hbm.at[pl.ds((g + 1) * CH, CH)], inbufs[(g + 1) % 2], sin)
                nxt.start()
            if g >= 2:
                pltpu.make_async_copy(
                    outb, out_hbm.at[pl.ds((g - 2) * CH, CH)], sout).wait()
            c = plsc.parallel_loop(0, CH, 1, unroll=16, carry=c)(
                make_row_step(inb, outb))
            st = pltpu.make_async_copy(outb, out_hbm.at[pl.ds(g * CH, CH)], sout)
            st.start()
            if g + 1 < NCH:
                nxt.wait()
        pltpu.make_async_copy(
            outbufs[(NCH - 2) % 2], out_hbm.at[pl.ds((NCH - 2) * CH, CH)], sout).wait()
        pltpu.make_async_copy(
            outbufs[(NCH - 1) % 2], out_hbm.at[pl.ds((NCH - 1) * CH, CH)], sout).wait()


_sc_scan = pl.kernel(
    _sc_scan_body,
    out_type=jax.ShapeDtypeStruct((N, M), jnp.float32),
    mesh=plsc.VectorSubcoreMesh(core_axis_name="c", subcore_axis_name="s", num_cores=1),
    scratch_types=[
        pltpu.VMEM((CH, M), jnp.float32),
        pltpu.VMEM((CH, M), jnp.float32),
        pltpu.VMEM((CH, M), jnp.float32),
        pltpu.VMEM((CH, M), jnp.float32),
        pltpu.SemaphoreType.DMA,
        pltpu.SemaphoreType.DMA,
    ],
)


def kernel(V):
    return _sc_scan(_tc_shift(V))


# speedup vs baseline: 1.0536x; 1.0536x over previous
import functools

import jax
import jax.numpy as jnp
from jax import lax
from jax.experimental import pallas as pl
from jax.experimental.pallas import tpu as pltpu
from jax.experimental.pallas import tpu_sc as plsc

N = 2048
M = 64
CH = 256
NCH = N // CH
CHW = CH * M


def _shift_body(v_ref, w_ref):
    v = v_ref[...]
    w_ref[...] = v - jnp.min(v, axis=1, keepdims=True) + 1.0


def _tc_shift(v):
    return pl.pallas_call(
        _shift_body,
        out_shape=jax.ShapeDtypeStruct((N, M), jnp.float32),
    )(v)


def _sc_scan_body(w_hbm, out_hbm, inb0, inb1, outb0, outb1, sin, sout):
    wid = lax.axis_index("s") * 2 + lax.axis_index("c")

    @pl.when(wid == 0)
    def _():
        ones = jnp.ones((16,), jnp.float32)
        lane = lax.iota(jnp.int32, 16)
        p8 = lane ^ 8
        p4 = lane ^ 4
        p2 = lane ^ 2
        p1 = lane ^ 1

        inbufs = (inb0, inb1)
        outbufs = (outb0, outb1)

        def make_row_step(inb, outb, nxt_inb, off=0):
            def row_step(i, carry):
                c0, c1, c2, c3, z0, z1, z2, z3 = carry
                e0 = jnp.exp(z0)
                e1 = jnp.exp(z1)
                e2 = jnp.exp(z2)
                e3 = jnp.exp(z3)
                s = (e0 + e1) + (e2 + e3)
                s = s + s.at[p8].get(mode="promise_in_bounds")
                s = s + s.at[p4].get(mode="promise_in_bounds")
                s = s + s.at[p2].get(mode="promise_in_bounds")
                s = s + s.at[p1].get(mode="promise_in_bounds")
                inv = 1.0 / s
                outb[i, pl.ds(0, 16)] = e0 * inv
                outb[i, pl.ds(16, 16)] = e1 * inv
                outb[i, pl.ds(32, 16)] = e2 * inv
                outb[i, pl.ds(48, 16)] = e3 * inv
                j = i + 1 - off
                a0 = nxt_inb[j, pl.ds(0, 16)] * c0
                a1 = nxt_inb[j, pl.ds(16, 16)] * c1
                a2 = nxt_inb[j, pl.ds(32, 16)] * c2
                a3 = nxt_inb[j, pl.ds(48, 16)] * c3
                return (c0 - (c0 * e0) * inv, c1 - (c1 * e1) * inv,
                        c2 - (c2 * e2) * inv, c3 - (c3 * e3) * inv,
                        a0 - (a0 * e0) * inv, a1 - (a1 * e1) * inv,
                        a2 - (a2 * e2) * inv, a3 - (a3 * e3) * inv)
            return row_step

        cp0 = pltpu.make_async_copy(w_hbm.at[pl.ds(0, CH)], inb0, sin)
        cp0.start()
        cp0.wait()

        c = (ones, ones, ones, ones,
             inb0[0, pl.ds(0, 16)], inb0[0, pl.ds(16, 16)],
             inb0[0, pl.ds(32, 16)], inb0[0, pl.ds(48, 16)])

        for g in range(NCH):
            inb = inbufs[g % 2]
            outb = outbufs[g % 2]
            if g + 1 < NCH:
                nxt = pltpu.make_async_copy(
                    w_hbm.at[pl.ds((g + 1) * CH, CH)], inbufs[(g + 1) % 2], sin)
                nxt.start()
            if g >= 2:
                pltpu.make_async_copy(
                    outb, out_hbm.at[pl.ds((g - 2) * CH, CH)], sout).wait()
            c = plsc.parallel_loop(0, CH - 1, 1, unroll=8, carry=c)(
                make_row_step(inb, outb, inb))
            if g + 1 < NCH:
                nxt.wait()
                nb = inbufs[(g + 1) % 2]
            else:
                nb = inb
            step_last = make_row_step(inb, outb, nb, off=CH)
            c = step_last(CH - 1, c)
            st = pltpu.make_async_copy(outb, out_hbm.at[pl.ds(g * CH, CH)], sout)
            st.start()
        pltpu.make_async_copy(
            outbufs[(NCH - 2) % 2], out_hbm.at[pl.ds((NCH - 2) * CH, CH)], sout).wait()
        pltpu.make_async_copy(
            outbufs[(NCH - 1) % 2], out_hbm.at[pl.ds((NCH - 1) * CH, CH)], sout).wait()


_sc_scan = pl.kernel(
    _sc_scan_body,
    out_type=jax.ShapeDtypeStruct((N, M), jnp.float32),
    mesh=plsc.VectorSubcoreMesh(core_axis_name="c", subcore_axis_name="s", num_cores=1),
    scratch_types=[
        pltpu.VMEM((CH, M), jnp.float32),
        pltpu.VMEM((CH, M), jnp.float32),
        pltpu.VMEM((CH, M), jnp.float32),
        pltpu.VMEM((CH, M), jnp.float32),
        pltpu.SemaphoreType.DMA,
        pltpu.SemaphoreType.DMA,
    ],
)


def kernel(V):
    return _sc_scan(_tc_shift(V))


# speedup vs baseline: 1.0538x; 1.0002x over previous
import functools

import jax
import jax.numpy as jnp
from jax import lax
from jax.experimental import pallas as pl
from jax.experimental.pallas import tpu as pltpu
from jax.experimental.pallas import tpu_sc as plsc

N = 2048
M = 64
CH = 256
NCH = N // CH
CHW = CH * M


def _shift_body(v_ref, w_ref):
    v = v_ref[...]
    w_ref[...] = v - jnp.min(v, axis=1, keepdims=True) + 1.0


def _tc_shift(v):
    return pl.pallas_call(
        _shift_body,
        out_shape=jax.ShapeDtypeStruct((N, M), jnp.float32),
    )(v)


def _sc_scan_body(w_hbm, out_hbm, inb0, inb1, outb0, outb1, sin, sout):
    wid = lax.axis_index("s") * 2 + lax.axis_index("c")

    @pl.when(wid == 0)
    def _():
        ones = jnp.ones((16,), jnp.float32)
        lane = lax.iota(jnp.int32, 16)
        p8 = lane ^ 8
        p4 = lane ^ 4
        p2 = lane ^ 2
        p1 = lane ^ 1

        inbufs = (inb0, inb1)
        outbufs = (outb0, outb1)

        def make_row_step(inb, outb, nxt_inb, off=0):
            def row_step(i, carry):
                c0, c1, c2, c3, z0, z1, z2, z3 = carry
                e0 = jnp.exp(z0)
                e1 = jnp.exp(z1)
                e2 = jnp.exp(z2)
                e3 = jnp.exp(z3)
                s = (e0 + e1) + (e2 + e3)
                s = s + s.at[p8].get(mode="promise_in_bounds")
                s = s + s.at[p4].get(mode="promise_in_bounds")
                s = s + s.at[p2].get(mode="promise_in_bounds")
                s = s + s.at[p1].get(mode="promise_in_bounds")
                inv = 1.0 / s
                outb[i, pl.ds(0, 16)] = e0 * inv
                outb[i, pl.ds(16, 16)] = e1 * inv
                outb[i, pl.ds(32, 16)] = e2 * inv
                outb[i, pl.ds(48, 16)] = e3 * inv
                j = i + 1 - off
                a0 = nxt_inb[j, pl.ds(0, 16)] * c0
                a1 = nxt_inb[j, pl.ds(16, 16)] * c1
                a2 = nxt_inb[j, pl.ds(32, 16)] * c2
                a3 = nxt_inb[j, pl.ds(48, 16)] * c3
                return (c0 - (c0 * e0) * inv, c1 - (c1 * e1) * inv,
                        c2 - (c2 * e2) * inv, c3 - (c3 * e3) * inv,
                        a0 - (a0 * e0) * inv, a1 - (a1 * e1) * inv,
                        a2 - (a2 * e2) * inv, a3 - (a3 * e3) * inv)
            return row_step

        cp0 = pltpu.make_async_copy(w_hbm.at[pl.ds(0, CH)], inb0, sin)
        cp0.start()
        cp0.wait()

        c = (ones, ones, ones, ones,
             inb0[0, pl.ds(0, 16)], inb0[0, pl.ds(16, 16)],
             inb0[0, pl.ds(32, 16)], inb0[0, pl.ds(48, 16)])

        for g in range(NCH):
            inb = inbufs[g % 2]
            outb = outbufs[g % 2]
            if g + 1 < NCH:
                nxt = pltpu.make_async_copy(
                    w_hbm.at[pl.ds((g + 1) * CH, CH)], inbufs[(g + 1) % 2], sin)
                nxt.start()
            if g >= 2:
                pltpu.make_async_copy(
                    outb, out_hbm.at[pl.ds((g - 2) * CH, CH)], sout).wait()
            c = plsc.parallel_loop(0, CH - 1, 1, unroll=16, carry=c)(
                make_row_step(inb, outb, inb))
            if g + 1 < NCH:
                nxt.wait()
                nb = inbufs[(g + 1) % 2]
            else:
                nb = inb
            step_last = make_row_step(inb, outb, nb, off=CH)
            c = step_last(CH - 1, c)
            st = pltpu.make_async_copy(outb, out_hbm.at[pl.ds(g * CH, CH)], sout)
            st.start()
        pltpu.make_async_copy(
            outbufs[(NCH - 2) % 2], out_hbm.at[pl.ds((NCH - 2) * CH, CH)], sout).wait()
        pltpu.make_async_copy(
            outbufs[(NCH - 1) % 2], out_hbm.at[pl.ds((NCH - 1) * CH, CH)], sout).wait()


_sc_scan = pl.kernel(
    _sc_scan_body,
    out_type=jax.ShapeDtypeStruct((N, M), jnp.float32),
    mesh=plsc.VectorSubcoreMesh(core_axis_name="c", subcore_axis_name="s", num_cores=1),
    scratch_types=[
        pltpu.VMEM((CH, M), jnp.float32),
        pltpu.VMEM((CH, M), jnp.float32),
        pltpu.VMEM((CH, M), jnp.float32),
        pltpu.VMEM((CH, M), jnp.float32),
        pltpu.SemaphoreType.DMA,
        pltpu.SemaphoreType.DMA,
    ],
)


def kernel(V):
    return _sc_scan(_tc_shift(V))
